# baseline (device time: 9533 ns/iter reference)
import jax
import jax.numpy as jnp
from jax import lax
from jax.experimental import pallas as pl
from jax.experimental.pallas import tpu as pltpu

N_DEV = 4
QSCALE = 127.0 / 96.0


def kernel(A, B):
    m, _ = A.shape
    _, n = B.shape
    m_out = m // N_DEV

    def body(a_any, b_any, out_ref, part_ref, send_ref, recv_ref,
             send_sems, recv_sems, entry_sems, a_ref, b_ref, copy_sems):
        p = lax.axis_index("i")
        left = (p - 1) % N_DEV
        right = (p + 1) % N_DEV
        diag = (p + 2) % N_DEV

        barrier = pltpu.get_barrier_semaphore()
        pl.semaphore_signal(
            barrier, inc=1,
            device_id=(diag,), device_id_type=pl.DeviceIdType.MESH,
        )
        pl.semaphore_signal(
            entry_sems.at[0], inc=1,
            device_id=(right,), device_id_type=pl.DeviceIdType.MESH,
        )
        pl.semaphore_signal(
            entry_sems.at[1], inc=1,
            device_id=(left,), device_id_type=pl.DeviceIdType.MESH,
        )

        cp_a = pltpu.make_async_copy(a_any, a_ref, copy_sems.at[0])
        cp_b = pltpu.make_async_copy(b_any, b_ref, copy_sems.at[1])
        cp_a.start()
        cp_b.start()
        cp_a.wait()
        cp_b.wait()

        part_ref[:, :] = jnp.dot(
            a_ref[:, :].astype(jnp.bfloat16),
            b_ref[:, :].astype(jnp.bfloat16),
            preferred_element_type=jnp.float32,
        )

        def chunk_partial(c):
            return part_ref[pl.ds(c * m_out, m_out), :]

        for d in (1, 2, 3):
            q = (p + d) % N_DEV
            scaled = chunk_partial(q) * QSCALE
            send_ref[d - 1, :, :] = jnp.clip(
                jnp.round(scaled), -127.0, 127.0
            ).astype(jnp.int8)

        def make_rdma(d):
            return pltpu.make_async_remote_copy(
                src_ref=send_ref.at[d - 1],
                dst_ref=recv_ref.at[N_DEV - 1 - d],
                send_sem=send_sems.at[d - 1],
                recv_sem=recv_sems.at[N_DEV - 1 - d],
                device_id=((p + d) % N_DEV,),
                device_id_type=pl.DeviceIdType.MESH,
            )

        rdmas = []
        for d, gate, count in (
            (1, entry_sems.at[1], None),
            (3, entry_sems.at[0], None),
            (2, barrier, 1),
        ):
            if count is None:
                pl.semaphore_wait(gate, 1)
            else:
                pl.semaphore_wait(gate, count)
            rdma = make_rdma(d)
            rdma.start()
            rdmas.append(rdma)

        acc = chunk_partial(p)
        for rdma, d in zip(rdmas, (1, 3, 2)):
            rdma.wait()
            acc = acc + recv_ref[N_DEV - 1 - d, :, :].astype(jnp.float32) * (
                1.0 / QSCALE
            )
        out_ref[:, :] = acc

    return pl.pallas_call(
        body,
        out_shape=jax.ShapeDtypeStruct((m_out, n), jnp.float32),
        in_specs=[
            pl.BlockSpec(memory_space=pl.ANY),
            pl.BlockSpec(memory_space=pl.ANY),
        ],
        out_specs=pl.BlockSpec(memory_space=pltpu.VMEM),
        scratch_shapes=[
            pltpu.VMEM((m, n), jnp.float32),
            pltpu.VMEM((N_DEV - 1, m_out, n), jnp.int8),
            pltpu.VMEM((N_DEV - 1, m_out, n), jnp.int8),
            pltpu.SemaphoreType.DMA((N_DEV - 1,)),
            pltpu.SemaphoreType.DMA((N_DEV - 1,)),
            pltpu.SemaphoreType.REGULAR((2,)),
            pltpu.VMEM(A.shape, jnp.float32),
            pltpu.VMEM(B.shape, jnp.float32),
            pltpu.SemaphoreType.DMA((2,)),
        ],
        compiler_params=pltpu.CompilerParams(collective_id=0),
    )(A, B)


# device time: 9407 ns/iter; 1.0134x vs baseline; 1.0134x over previous
import jax
import jax.numpy as jnp
from jax import lax
from jax.experimental import pallas as pl
from jax.experimental.pallas import tpu as pltpu

N_DEV = 4
QSCALE = 127.0 / 96.0


def kernel(A, B):
    m, _ = A.shape
    _, n = B.shape
    m_out = m // N_DEV

    def body(a_ref, b_ref, out_ref, part_ref, send_ref, recv_ref,
             send_sems, recv_sems, entry_sems):
        p = lax.axis_index("i")
        left = (p - 1) % N_DEV
        right = (p + 1) % N_DEV
        diag = (p + 2) % N_DEV

        barrier = pltpu.get_barrier_semaphore()
        pl.semaphore_signal(
            barrier, inc=1,
            device_id=(diag,), device_id_type=pl.DeviceIdType.MESH,
        )
        pl.semaphore_signal(
            entry_sems.at[0], inc=1,
            device_id=(right,), device_id_type=pl.DeviceIdType.MESH,
        )
        pl.semaphore_signal(
            entry_sems.at[1], inc=1,
            device_id=(left,), device_id_type=pl.DeviceIdType.MESH,
        )

        part_ref[:, :] = jnp.dot(
            a_ref[:, :].astype(jnp.bfloat16),
            b_ref[:, :].astype(jnp.bfloat16),
            preferred_element_type=jnp.float32,
        )

        def chunk_partial(c):
            return part_ref[pl.ds(c * m_out, m_out), :]

        for d in (1, 2, 3):
            q = (p + d) % N_DEV
            scaled = chunk_partial(q) * QSCALE
            send_ref[d - 1, :, :] = jnp.clip(
                jnp.round(scaled), -127.0, 127.0
            ).astype(jnp.int8)

        def make_rdma(d):
            return pltpu.make_async_remote_copy(
                src_ref=send_ref.at[d - 1],
                dst_ref=recv_ref.at[N_DEV - 1 - d],
                send_sem=send_sems.at[d - 1],
                recv_sem=recv_sems.at[N_DEV - 1 - d],
                device_id=((p + d) % N_DEV,),
                device_id_type=pl.DeviceIdType.MESH,
            )

        rdmas = []
        for d, gate, count in (
            (1, entry_sems.at[1], None),
            (3, entry_sems.at[0], None),
            (2, barrier, 1),
        ):
            if count is None:
                pl.semaphore_wait(gate, 1)
            else:
                pl.semaphore_wait(gate, count)
            rdma = make_rdma(d)
            rdma.start()
            rdmas.append(rdma)

        acc = chunk_partial(p)
        for rdma, d in zip(rdmas, (1, 3, 2)):
            rdma.wait()
            acc = acc + recv_ref[N_DEV - 1 - d, :, :].astype(jnp.float32) * (
                1.0 / QSCALE
            )
        out_ref[:, :] = acc

    return pl.pallas_call(
        body,
        out_shape=jax.ShapeDtypeStruct((m_out, n), jnp.float32),
        in_specs=[
            pl.BlockSpec(memory_space=pltpu.VMEM),
            pl.BlockSpec(memory_space=pltpu.VMEM),
        ],
        out_specs=pl.BlockSpec(memory_space=pltpu.VMEM),
        scratch_shapes=[
            pltpu.VMEM((m, n), jnp.float32),
            pltpu.VMEM((N_DEV - 1, m_out, n), jnp.int8),
            pltpu.VMEM((N_DEV - 1, m_out, n), jnp.int8),
            pltpu.SemaphoreType.DMA((N_DEV - 1,)),
            pltpu.SemaphoreType.DMA((N_DEV - 1,)),
            pltpu.SemaphoreType.REGULAR((2,)),
        ],
        compiler_params=pltpu.CompilerParams(collective_id=0),
    )(A, B)
